# baseline (speedup 1.0000x reference)
import functools

import jax
import jax.numpy as jnp
from jax import lax
from jax.experimental import pallas as pl
from jax.experimental.pallas import tpu as pltpu
from jax.experimental.pallas import tpu_sc as plsc

NC = 2
NS = 16
L = 16
NW = NC * NS
B = 16384
D = 64
CHUNK = 128
CPW = B // NW // CHUNK
NROW = B // CHUNK

_mesh = plsc.VectorSubcoreMesh(
    core_axis_name="c", subcore_axis_name="s", num_cores=NC, num_subcores=NS
)


@functools.partial(
    pl.kernel,
    out_type=(
        jax.ShapeDtypeStruct((NW * L,), jnp.float32),
        jax.ShapeDtypeStruct((NROW, CHUNK), jnp.float32),
        jax.ShapeDtypeStruct((NROW, CHUNK), jnp.float32),
    ),
    mesh=_mesh,
    scratch_types=[
        pltpu.VMEM((CPW, CHUNK), jnp.int32),
        pltpu.VMEM((CPW, CHUNK), jnp.int32),
        pltpu.VMEM((CPW, CHUNK, D), jnp.float32),
        pltpu.VMEM((CPW, CHUNK, D), jnp.float32),
        pltpu.VMEM((CPW, CHUNK), jnp.float32),
        pltpu.VMEM((CPW, CHUNK), jnp.float32),
        pltpu.VMEM((L,), jnp.float32),
        pltpu.SemaphoreType.DMA,
    ],
    compiler_params=pltpu.CompilerParams(use_tc_tiling_on_sc=False),
)
def _gather_reduce(idxu_hbm, idxb_hbm, uemb_hbm, ubias_hbm, bemb_hbm, bbias_hbm,
                   part_hbm, ubg_hbm, bbg_hbm,
                   idxu_v, idxb_v, urows, brows, ubv, bbv, accv, sem):
    wid = lax.axis_index("s") * NC + lax.axis_index("c")
    base = wid * CPW
    pltpu.sync_copy(idxu_hbm.at[pl.ds(base, CPW)], idxu_v)
    pltpu.sync_copy(idxb_hbm.at[pl.ds(base, CPW)], idxb_v)

    copies = []
    for j in range(CPW):
        copies.append(pltpu.async_copy(uemb_hbm.at[idxu_v.at[j]], urows.at[j], sem))
        copies.append(pltpu.async_copy(bemb_hbm.at[idxb_v.at[j]], brows.at[j], sem))
        copies.append(pltpu.async_copy(ubias_hbm.at[idxu_v.at[j]], ubv.at[j], sem))
        copies.append(pltpu.async_copy(bbias_hbm.at[idxb_v.at[j]], bbv.at[j], sem))
    for c in copies:
        c.wait()

    acc = jnp.zeros((L,), jnp.float32)
    for j in range(CPW):
        def body(i, a, j=j):
            for c in range(D // L):
                a = a + urows[j, i, pl.ds(c * L, L)] * brows[j, i, pl.ds(c * L, L)]
            return a
        acc = lax.fori_loop(0, CHUNK, body, acc)
    accv[...] = acc

    pltpu.sync_copy(accv, part_hbm.at[pl.ds(wid * L, L)])
    pltpu.sync_copy(ubv, ubg_hbm.at[pl.ds(base, CPW)])
    pltpu.sync_copy(bbv, bbg_hbm.at[pl.ds(base, CPW)])


def _finish_body(p_ref, ub_ref, bb_ref, o_ref):
    s = jnp.sum(p_ref[...])
    x = s + ub_ref[...] + bb_ref[...]
    o_ref[...] = 1.0 / (1.0 + jnp.exp(-x))


def kernel(inputs, user_emb_table, user_bias_table, blog_emb_table, blog_bias_table):
    idx = inputs.astype(jnp.int32)
    idxu = idx[:, 0].reshape(NROW, CHUNK)
    idxb = idx[:, 1].reshape(NROW, CHUNK)
    part, ubg, bbg = _gather_reduce(
        idxu, idxb,
        user_emb_table, user_bias_table.reshape(-1),
        blog_emb_table, blog_bias_table.reshape(-1),
    )
    out = pl.pallas_call(
        _finish_body,
        out_shape=jax.ShapeDtypeStruct((NROW, CHUNK), jnp.float32),
    )(part.reshape(NW * L // CHUNK, CHUNK), ubg, bbg)
    return out.reshape(B, 1)
